# baseline (speedup 1.0000x reference)
import functools
import math

import jax
import jax.numpy as jnp
from jax import lax
from jax.experimental import pallas as pl
from jax.experimental.pallas import tpu as pltpu
from jax.experimental.pallas import tpu_sc as plsc

DIM = 64
SCALE = math.sqrt(DIM)
NUM_CORES = 2
NUM_SUBCORES = 16
NUM_WORKERS = NUM_CORES * NUM_SUBCORES
LANES = 16

CHUNK = 512
GATHER_SUB = 128


def _emb_kernel(num_rows):
    b_per_w = num_rows // NUM_WORKERS
    n_chunks = b_per_w // CHUNK
    mesh = plsc.VectorSubcoreMesh(core_axis_name="c", subcore_axis_name="s")

    @functools.partial(
        pl.kernel,
        mesh=mesh,
        out_type=jax.ShapeDtypeStruct((num_rows, DIM), jnp.float32),
        scratch_types=[
            pltpu.VMEM((CHUNK,), jnp.int32),
            pltpu.VMEM((CHUNK, DIM), jnp.float32),
            pltpu.SemaphoreType.DMA,
        ],
        compiler_params=pltpu.CompilerParams(use_tc_tiling_on_sc=False),
    )
    def body(idx_hbm, table_hbm, out_hbm, idx_v, rows_v, sem):
        wid = lax.axis_index("s") * NUM_CORES + lax.axis_index("c")
        base = wid * b_per_w

        def step(c, carry):
            off = base + c * CHUNK
            pltpu.sync_copy(idx_hbm.at[pl.ds(off, CHUNK)], idx_v)
            copies = [
                pltpu.async_copy(
                    table_hbm.at[idx_v.at[pl.ds(j * GATHER_SUB, GATHER_SUB)]],
                    rows_v.at[pl.ds(j * GATHER_SUB, GATHER_SUB)],
                    sem,
                )
                for j in range(CHUNK // GATHER_SUB)
            ]
            for cp in copies:
                cp.wait()

            def scale_rows(i, carry2):
                for g in range(DIM // LANES):
                    sl = (i, pl.ds(g * LANES, LANES))
                    rows_v[sl] = rows_v[sl] * SCALE
                return carry2

            lax.fori_loop(0, CHUNK, scale_rows, 0, unroll=4)
            pltpu.sync_copy(rows_v, out_hbm.at[pl.ds(off, CHUNK)])
            return carry

        lax.fori_loop(0, n_chunks, step, 0)

    return body


def kernel(x, table):
    num_rows = x.size
    idx = jnp.reshape(x, (num_rows,)).astype(jnp.int32)
    out = _emb_kernel(num_rows)(idx, table)
    return jnp.reshape(out, x.shape + (DIM,))


# speedup vs baseline: 1.0906x; 1.0906x over previous
import functools
import math

import jax
import jax.numpy as jnp
from jax import lax
from jax.experimental import pallas as pl
from jax.experimental.pallas import tpu as pltpu
from jax.experimental.pallas import tpu_sc as plsc

DIM = 64
SCALE = math.sqrt(DIM)
NUM_CORES = 2
NUM_SUBCORES = 16
NUM_WORKERS = NUM_CORES * NUM_SUBCORES
LANES = 16

CHUNK = 640
GATHER_SUB = 128
NSUB = CHUNK // GATHER_SUB


def _emb_kernel(num_rows):
    b_per_w = num_rows // NUM_WORKERS
    n_chunks = b_per_w // CHUNK
    mesh = plsc.VectorSubcoreMesh(core_axis_name="c", subcore_axis_name="s")

    @functools.partial(
        pl.kernel,
        mesh=mesh,
        out_type=jax.ShapeDtypeStruct((num_rows, DIM), jnp.float32),
        scratch_types=[
            pltpu.VMEM((b_per_w,), jnp.int32),
            pltpu.VMEM((CHUNK, DIM), jnp.float32),
            pltpu.VMEM((CHUNK, DIM), jnp.float32),
            pltpu.SemaphoreType.DMA,
            pltpu.SemaphoreType.DMA,
        ],
        compiler_params=pltpu.CompilerParams(use_tc_tiling_on_sc=False),
    )
    def body(idx_hbm, table_hbm, out_hbm, idx_v, rows_a, rows_b, gsem, ssem):
        wid = lax.axis_index("s") * NUM_CORES + lax.axis_index("c")
        base = wid * b_per_w
        bufs = (rows_a, rows_b)

        pltpu.sync_copy(idx_hbm.at[pl.ds(base, b_per_w)], idx_v)

        def fire_gather(c, buf):
            for j in range(NSUB):
                pltpu.async_copy(
                    table_hbm.at[idx_v.at[pl.ds(c * CHUNK + j * GATHER_SUB,
                                                GATHER_SUB)]],
                    buf.at[pl.ds(j * GATHER_SUB, GATHER_SUB)],
                    gsem,
                )

        def drain_gather(buf):
            for j in range(NSUB):
                pltpu.make_async_copy(
                    table_hbm.at[idx_v.at[pl.ds(j * GATHER_SUB, GATHER_SUB)]],
                    buf.at[pl.ds(j * GATHER_SUB, GATHER_SUB)],
                    gsem,
                ).wait()

        def wait_store(buf):
            pltpu.make_async_copy(
                buf, out_hbm.at[pl.ds(base, CHUNK)], ssem
            ).wait()

        fire_gather(0, bufs[0])

        def pair(p, carry):
            for b in range(2):
                c = p * 2 + b
                buf = bufs[b]

                @pl.when(c >= 1)
                def _():
                    wait_store(bufs[1 - b])

                @pl.when(c + 1 < n_chunks)
                def _():
                    fire_gather(c + 1, bufs[1 - b])

                drain_gather(buf)

                def scale_rows(i, carry2):
                    for g in range(DIM // LANES):
                        sl = (i, pl.ds(g * LANES, LANES))
                        buf[sl] = buf[sl] * SCALE
                    return carry2

                lax.fori_loop(0, CHUNK, scale_rows, 0, unroll=8)

                pltpu.async_copy(
                    buf, out_hbm.at[pl.ds(base + c * CHUNK, CHUNK)], ssem
                )
            return carry

        lax.fori_loop(0, n_chunks // 2, pair, 0)
        wait_store(bufs[(n_chunks - 1) % 2])

    return body


def kernel(x, table):
    num_rows = x.size
    idx = jnp.reshape(x, (num_rows,)).astype(jnp.int32)
    out = _emb_kernel(num_rows)(idx, table)
    return jnp.reshape(out, x.shape + (DIM,))
